# baseline (speedup 1.0000x reference)
import functools

import jax
import jax.numpy as jnp
from jax import lax
from jax.experimental import pallas as pl
from jax.experimental.pallas import tpu as pltpu
from jax.experimental.pallas import tpu_sc as plsc

BATCH = 16384
EMB_DIM = 64
LANES = 16


def _scalar(vec, l):
    return lax.squeeze(lax.slice(vec, (l,), (l + 1,)), dimensions=(0,))


def kernel(head, relation, tail, ent_emb, rel_emb):
    head = head.reshape(-1).astype(jnp.int32)
    rel = relation.reshape(-1).astype(jnp.int32)
    tail = tail.reshape(-1).astype(jnp.int32)
    rel_flat = rel_emb.reshape(-1)
    n_rel_words = rel_flat.shape[0]

    info = plsc.get_sparse_core_info()
    nw = info.num_cores * info.num_subcores
    b_per_w = BATCH // nw
    n_groups = b_per_w // LANES

    mesh = plsc.VectorSubcoreMesh(core_axis_name="c", subcore_axis_name="s")

    @functools.partial(
        pl.kernel,
        mesh=mesh,
        out_type=jax.ShapeDtypeStruct((BATCH * EMB_DIM,), jnp.float32),
        scratch_types=[
            pltpu.VMEM((b_per_w,), jnp.int32),
            pltpu.VMEM((b_per_w,), jnp.int32),
            pltpu.VMEM((b_per_w,), jnp.int32),
            pltpu.VMEM((LANES, EMB_DIM), jnp.float32),
            pltpu.VMEM((LANES, EMB_DIM), jnp.float32),
            pltpu.VMEM((LANES, EMB_DIM), jnp.float32),
            pltpu.VMEM((LANES, EMB_DIM), jnp.float32),
            pltpu.VMEM((n_rel_words,), jnp.float32),
            pltpu.VMEM((b_per_w * EMB_DIM,), jnp.float32),
            pltpu.SemaphoreType.DMA,
            pltpu.SemaphoreType.DMA,
        ],
    )
    def trans_e(head_hbm, rel_hbm, tail_hbm, ent_hbm, relflat_hbm, out_hbm,
                hidx, ridx, tidx, hbuf0, hbuf1, tbuf0, tbuf1, rtab, obuf,
                sem0, sem1):
        wid = lax.axis_index("s") * info.num_cores + lax.axis_index("c")
        base = wid * b_per_w

        pltpu.sync_copy(head_hbm.at[pl.ds(base, b_per_w)], hidx)
        pltpu.sync_copy(rel_hbm.at[pl.ds(base, b_per_w)], ridx)
        pltpu.sync_copy(tail_hbm.at[pl.ds(base, b_per_w)], tidx)
        pltpu.sync_copy(relflat_hbm, rtab)

        def fire(g, hb, tb, sem):
            gs = pl.ds(g * LANES, LANES)
            hch = hidx[gs]
            tch = tidx[gs]
            for l in range(LANES):
                hs = _scalar(hch, l)
                ts = _scalar(tch, l)
                pltpu.async_copy(ent_hbm.at[hs], hb.at[l], sem)
                pltpu.async_copy(ent_hbm.at[ts], tb.at[l], sem)

        def drain(hb, tb, sem):
            for l in range(LANES):
                pltpu.make_async_copy(ent_hbm.at[0], hb.at[l], sem).wait()
                pltpu.make_async_copy(ent_hbm.at[0], tb.at[l], sem).wait()

        def compute(g, hb, tb):
            gs = pl.ds(g * LANES, LANES)
            rch = ridx[gs]
            for l in range(LANES):
                rbase = _scalar(rch, l) * EMB_DIM
                ebase = (g * LANES + l) * EMB_DIM
                for k in range(EMB_DIM // LANES):
                    s = pl.ds(k * LANES, LANES)
                    os_ = pl.ds(ebase + k * LANES, LANES)
                    rs_ = pl.ds(rbase + k * LANES, LANES)
                    obuf[os_] = hb[l, s] + rtab[rs_] - tb[l, s]

        fire(0, hbuf0, tbuf0, sem0)

        def pair_body(p, carry):
            g0 = p * 2
            fire(g0 + 1, hbuf1, tbuf1, sem1)
            drain(hbuf0, tbuf0, sem0)
            compute(g0, hbuf0, tbuf0)

            @pl.when(p < n_groups // 2 - 1)
            def _():
                fire(g0 + 2, hbuf0, tbuf0, sem0)

            drain(hbuf1, tbuf1, sem1)
            compute(g0 + 1, hbuf1, tbuf1)
            return carry

        lax.fori_loop(0, n_groups // 2, pair_body, 0)

        pltpu.sync_copy(obuf, out_hbm.at[pl.ds(base * EMB_DIM, b_per_w * EMB_DIM)])

    out = trans_e(head, rel, tail, ent_emb, rel_flat)
    return out.reshape(BATCH, EMB_DIM)


# speedup vs baseline: 1.4385x; 1.4385x over previous
import functools

import jax
import jax.numpy as jnp
from jax import lax
from jax.experimental import layout as jlayout
from jax.experimental import pallas as pl
from jax.experimental.pallas import tpu as pltpu
from jax.experimental.pallas import tpu_sc as plsc

BATCH = 16384
EMB_DIM = 64
LANES = 16


def _scalar(vec, l):
    return lax.squeeze(lax.slice(vec, (l,), (l + 1,)), dimensions=(0,))


def kernel(head, relation, tail, ent_emb, rel_emb):
    head = head.reshape(-1).astype(jnp.int32)
    rel = relation.reshape(-1).astype(jnp.int32)
    tail = tail.reshape(-1).astype(jnp.int32)
    rel_flat = rel_emb.reshape(-1)
    n_rel_words = rel_flat.shape[0]
    n_ent = ent_emb.shape[0]

    row_major = jlayout.Format(
        jlayout.Layout((0, 1), tiling=((8, 128),)),
        jax.sharding.SingleDeviceSharding(jax.devices()[0]),
    )
    ent3 = jax.device_put(ent_emb, row_major).reshape(n_ent // 8, 8, EMB_DIM)

    info = plsc.get_sparse_core_info()
    nw = info.num_cores * info.num_subcores
    b_per_w = BATCH // nw
    n_groups = b_per_w // LANES

    mesh = plsc.VectorSubcoreMesh(core_axis_name="c", subcore_axis_name="s")

    @functools.partial(
        pl.kernel,
        mesh=mesh,
        out_type=jax.ShapeDtypeStruct((BATCH * EMB_DIM,), jnp.float32),
        scratch_types=[
            pltpu.VMEM((b_per_w,), jnp.int32),
            pltpu.VMEM((b_per_w,), jnp.int32),
            pltpu.VMEM((b_per_w,), jnp.int32),
            pltpu.VMEM((LANES, EMB_DIM), jnp.float32),
            pltpu.VMEM((LANES, EMB_DIM), jnp.float32),
            pltpu.VMEM((LANES, EMB_DIM), jnp.float32),
            pltpu.VMEM((LANES, EMB_DIM), jnp.float32),
            pltpu.VMEM((n_rel_words,), jnp.float32),
            pltpu.VMEM((b_per_w * EMB_DIM,), jnp.float32),
            pltpu.SemaphoreType.DMA,
            pltpu.SemaphoreType.DMA,
        ],
    )
    def trans_e(head_hbm, rel_hbm, tail_hbm, ent_hbm, relflat_hbm, out_hbm,
                hidx, ridx, tidx, hbuf0, hbuf1, tbuf0, tbuf1, rtab, obuf,
                sem0, sem1):
        wid = lax.axis_index("s") * info.num_cores + lax.axis_index("c")
        base = wid * b_per_w

        pltpu.sync_copy(head_hbm.at[pl.ds(base, b_per_w)], hidx)
        pltpu.sync_copy(rel_hbm.at[pl.ds(base, b_per_w)], ridx)
        pltpu.sync_copy(tail_hbm.at[pl.ds(base, b_per_w)], tidx)
        pltpu.sync_copy(relflat_hbm, rtab)

        def fire(g, hb, tb, sem):
            gs = pl.ds(g * LANES, LANES)
            hch = hidx[gs]
            tch = tidx[gs]
            for l in range(LANES):
                hs = _scalar(hch, l)
                ts = _scalar(tch, l)
                pltpu.async_copy(
                    ent_hbm.at[lax.shift_right_logical(hs, 3),
                               lax.bitwise_and(hs, 7)], hb.at[l], sem)
                pltpu.async_copy(
                    ent_hbm.at[lax.shift_right_logical(ts, 3),
                               lax.bitwise_and(ts, 7)], tb.at[l], sem)

        def drain(hb, tb, sem):
            for l in range(LANES):
                pltpu.make_async_copy(ent_hbm.at[0, 0], hb.at[l], sem).wait()
                pltpu.make_async_copy(ent_hbm.at[0, 0], tb.at[l], sem).wait()

        def compute(g, hb, tb):
            gs = pl.ds(g * LANES, LANES)
            rch = ridx[gs]
            for l in range(LANES):
                rbase = _scalar(rch, l) * EMB_DIM
                ebase = (g * LANES + l) * EMB_DIM
                for k in range(EMB_DIM // LANES):
                    s = pl.ds(k * LANES, LANES)
                    os_ = pl.ds(ebase + k * LANES, LANES)
                    rs_ = pl.ds(rbase + k * LANES, LANES)
                    obuf[os_] = hb[l, s] + rtab[rs_] - tb[l, s]

        fire(0, hbuf0, tbuf0, sem0)

        def pair_body(p, carry):
            g0 = p * 2
            fire(g0 + 1, hbuf1, tbuf1, sem1)
            drain(hbuf0, tbuf0, sem0)
            compute(g0, hbuf0, tbuf0)

            @pl.when(p < n_groups // 2 - 1)
            def _():
                fire(g0 + 2, hbuf0, tbuf0, sem0)

            drain(hbuf1, tbuf1, sem1)
            compute(g0 + 1, hbuf1, tbuf1)
            return carry

        lax.fori_loop(0, n_groups // 2, pair_body, 0)

        pltpu.sync_copy(obuf, out_hbm.at[pl.ds(base * EMB_DIM, b_per_w * EMB_DIM)])

    out = trans_e(head, rel, tail, ent3, rel_flat)
    return out.reshape(BATCH, EMB_DIM)
